# Initial kernel scaffold; baseline (speedup 1.0000x reference)
import functools

import jax
import jax.numpy as jnp
from jax import lax
from jax.experimental import pallas as pl
from jax.experimental.pallas import tpu as pltpu
from jax.experimental.pallas import tpu_sc as plsc

EMBED_DIM = 64

_info = plsc.get_sparse_core_info()
_NC = _info.num_cores
_NS = _info.num_subcores
_NW = _NC * _NS

_SUB = 128
_CHUNK = 512
_SUB_PER_CHUNK = _CHUNK // _SUB


def _make_embed(b_total):
    assert b_total % (_NW * _CHUNK) == 0
    b_per_w = b_total // _NW
    n_chunks = b_per_w // _CHUNK
    mesh = plsc.VectorSubcoreMesh(core_axis_name="c", subcore_axis_name="s")

    @functools.partial(
        pl.kernel,
        mesh=mesh,
        out_type=jax.ShapeDtypeStruct((b_total, EMBED_DIM), jnp.float32),
        scratch_types=[
            pltpu.VMEM((_SUB_PER_CHUNK, _SUB), jnp.int32),
            pltpu.VMEM((_CHUNK, EMBED_DIM), jnp.float32),
            pltpu.SemaphoreType.DMA,
        ],
    )
    def embed(table_hbm, idx_hbm, out_hbm, idx_v, rows_v, sem):
        wid = lax.axis_index("s") * _NC + lax.axis_index("c")
        base = wid * b_per_w

        def body(i, carry):
            off = base + i * _CHUNK
            pltpu.sync_copy(idx_hbm.at[pl.ds(off, _CHUNK)], idx_v)
            copies = []
            for j in range(_SUB_PER_CHUNK):
                copies.append(
                    pltpu.async_copy(
                        table_hbm.at[idx_v.at[j]],
                        rows_v.at[pl.ds(j * _SUB, _SUB)],
                        sem,
                    )
                )
            for c in copies:
                c.wait()
            pltpu.sync_copy(rows_v, out_hbm.at[pl.ds(off, _CHUNK)])
            return carry

        lax.fori_loop(0, n_chunks, body, 0)

    return embed


def kernel(tokens, table):
    flat = tokens.reshape(-1)
    out = _make_embed(flat.shape[0])(table, flat)
    return out.reshape(tokens.shape + (EMBED_DIM,))


# speedup vs baseline: 3.9570x; 3.9570x over previous
import functools

import jax
import jax.numpy as jnp
from jax import lax
from jax.experimental import pallas as pl
from jax.experimental.pallas import tpu as pltpu
from jax.experimental.pallas import tpu_sc as plsc

EMBED_DIM = 64

_info = plsc.get_sparse_core_info()
_NC = _info.num_cores
_NS = _info.num_subcores
_NW = _NC * _NS

_SUB = 128
_CHUNK = 512
_SUB_PER_CHUNK = _CHUNK // _SUB


def _make_embed(b_total):
    assert b_total % (_NW * _CHUNK) == 0
    b_per_w = b_total // _NW
    n_chunks = b_per_w // _CHUNK
    mesh = plsc.VectorSubcoreMesh(core_axis_name="c", subcore_axis_name="s")

    @functools.partial(
        pl.kernel,
        mesh=mesh,
        out_type=jax.ShapeDtypeStruct((b_total, EMBED_DIM), jnp.float32),
        scratch_types=[
            pltpu.VMEM((_SUB_PER_CHUNK, _SUB), jnp.int32),
            pltpu.VMEM((_CHUNK, EMBED_DIM), jnp.float32),
            pltpu.SemaphoreType.DMA,
        ],
        compiler_params=pltpu.CompilerParams(use_tc_tiling_on_sc=False),
    )
    def embed(table_hbm, idx_hbm, out_hbm, idx_v, rows_v, sem):
        wid = lax.axis_index("s") * _NC + lax.axis_index("c")
        base = wid * b_per_w
        base_row = wid * (b_per_w // _SUB)

        def body(i, carry):
            off = base + i * _CHUNK
            row_off = base_row + i * _SUB_PER_CHUNK
            pltpu.sync_copy(idx_hbm.at[pl.ds(row_off, _SUB_PER_CHUNK)], idx_v)
            copies = []
            for j in range(_SUB_PER_CHUNK):
                copies.append(
                    pltpu.async_copy(
                        table_hbm.at[idx_v.at[j]],
                        rows_v.at[pl.ds(j * _SUB, _SUB)],
                        sem,
                    )
                )
            for c in copies:
                c.wait()
            pltpu.sync_copy(rows_v, out_hbm.at[pl.ds(off, _CHUNK)])
            return carry

        lax.fori_loop(0, n_chunks, body, 0)

    return embed


def kernel(tokens, table):
    flat = tokens.reshape(-1)
    idx2d = flat.reshape(-1, _SUB)
    out = _make_embed(flat.shape[0])(table, idx2d)
    return out.reshape(tokens.shape + (EMBED_DIM,))


# speedup vs baseline: 4.2336x; 1.0699x over previous
import functools

import jax
import jax.numpy as jnp
from jax import lax
from jax.experimental import pallas as pl
from jax.experimental.pallas import tpu as pltpu
from jax.experimental.pallas import tpu_sc as plsc

EMBED_DIM = 64

_info = plsc.get_sparse_core_info()
_NC = _info.num_cores
_NS = _info.num_subcores
_NW = _NC * _NS

_SUB = 128
_CHUNK = 512
_SUB_PER_CHUNK = _CHUNK // _SUB


def _make_embed(b_total):
    assert b_total % (_NW * 2 * _CHUNK) == 0
    b_per_w = b_total // _NW
    n_chunks = b_per_w // _CHUNK
    n_pairs = n_chunks // 2
    mesh = plsc.VectorSubcoreMesh(core_axis_name="c", subcore_axis_name="s")

    @functools.partial(
        pl.kernel,
        mesh=mesh,
        out_type=jax.ShapeDtypeStruct((b_total, EMBED_DIM), jnp.float32),
        scratch_types=[
            pltpu.VMEM((2, _SUB_PER_CHUNK, _SUB), jnp.int32),
            pltpu.VMEM((2, _CHUNK, EMBED_DIM), jnp.float32),
            pltpu.SemaphoreType.DMA,
            pltpu.SemaphoreType.DMA,
            pltpu.SemaphoreType.DMA,
            pltpu.SemaphoreType.DMA,
            pltpu.SemaphoreType.DMA,
        ],
        compiler_params=pltpu.CompilerParams(use_tc_tiling_on_sc=False),
    )
    def embed(table_hbm, idx_hbm, out_hbm, idx_v, rows_v, sem_g,
              sem_i0, sem_i1, sem_o0, sem_o1):
        sem_i = [sem_i0, sem_i1]
        sem_o = [sem_o0, sem_o1]
        wid = lax.axis_index("s") * _NC + lax.axis_index("c")
        base = wid * b_per_w
        base_row = wid * (b_per_w // _SUB)

        def start_idx(c, b):
            pltpu.async_copy(
                idx_hbm.at[pl.ds(base_row + c * _SUB_PER_CHUNK, _SUB_PER_CHUNK)],
                idx_v.at[b],
                sem_i[b],
            )

        def wait_out(b):
            pltpu.make_async_copy(
                rows_v.at[b], out_hbm.at[pl.ds(base, _CHUNK)], sem_o[b]
            ).wait()

        def process(c, b, wait_prev_out, prefetch_idx):
            if wait_prev_out:
                wait_out(b)
            pltpu.make_async_copy(
                idx_hbm.at[pl.ds(base_row, _SUB_PER_CHUNK)],
                idx_v.at[b],
                sem_i[b],
            ).wait()
            copies = []
            for j in range(_SUB_PER_CHUNK):
                copies.append(
                    pltpu.async_copy(
                        table_hbm.at[idx_v.at[b].at[j]],
                        rows_v.at[b].at[pl.ds(j * _SUB, _SUB)],
                        sem_g,
                    )
                )
            for cp in copies:
                cp.wait()
            if prefetch_idx:
                start_idx(c + 2, b)
            pltpu.async_copy(
                rows_v.at[b],
                out_hbm.at[pl.ds(base + c * _CHUNK, _CHUNK)],
                sem_o[b],
            )

        start_idx(0, 0)
        start_idx(1, 1)
        process(0, 0, wait_prev_out=False, prefetch_idx=True)
        process(1, 1, wait_prev_out=False, prefetch_idx=True)

        def body(g, carry):
            c = 2 * g
            process(c, 0, wait_prev_out=True, prefetch_idx=True)
            process(c + 1, 1, wait_prev_out=True, prefetch_idx=True)
            return carry

        lax.fori_loop(1, n_pairs - 1, body, 0)

        c = 2 * (n_pairs - 1)
        process(c, 0, wait_prev_out=True, prefetch_idx=False)
        process(c + 1, 1, wait_prev_out=True, prefetch_idx=False)

        wait_out(0)
        wait_out(1)

    return embed


def kernel(tokens, table):
    flat = tokens.reshape(-1)
    idx2d = flat.reshape(-1, _SUB)
    out = _make_embed(flat.shape[0])(table, idx2d)
    return out.reshape(tokens.shape + (EMBED_DIM,))


# speedup vs baseline: 4.2410x; 1.0018x over previous
import functools

import jax
import jax.numpy as jnp
from jax import lax
from jax.experimental import pallas as pl
from jax.experimental.pallas import tpu as pltpu
from jax.experimental.pallas import tpu_sc as plsc

EMBED_DIM = 64
SEQ = 200

_info = plsc.get_sparse_core_info()
_NC = _info.num_cores
_NS = _info.num_subcores
_NW = _NC * _NS

_IDXROW = 100
_IPB = SEQ // _IDXROW
_CB = 4


def _make_embed(batch):
    assert batch % (_NW * 2 * _CB) == 0
    b_per_w = batch // _NW
    n_chunks = b_per_w // _CB
    n_pairs = n_chunks // 2
    mesh = plsc.VectorSubcoreMesh(core_axis_name="c", subcore_axis_name="s")

    @functools.partial(
        pl.kernel,
        mesh=mesh,
        out_type=jax.ShapeDtypeStruct((batch, SEQ, EMBED_DIM), jnp.float32),
        scratch_types=[
            pltpu.VMEM((2, _CB * _IPB, _IDXROW), jnp.int32),
            pltpu.VMEM((2, _CB, SEQ, EMBED_DIM), jnp.float32),
            pltpu.SemaphoreType.DMA,
            pltpu.SemaphoreType.DMA,
            pltpu.SemaphoreType.DMA,
            pltpu.SemaphoreType.DMA,
            pltpu.SemaphoreType.DMA,
        ],
        compiler_params=pltpu.CompilerParams(use_tc_tiling_on_sc=False),
    )
    def embed(table_hbm, idx_hbm, out_hbm, idx_v, rows_v, sem_g,
              sem_i0, sem_i1, sem_o0, sem_o1):
        sem_i = [sem_i0, sem_i1]
        sem_o = [sem_o0, sem_o1]
        wid = lax.axis_index("s") * _NC + lax.axis_index("c")
        base = wid * b_per_w

        def start_idx(c, b):
            pltpu.async_copy(
                idx_hbm.at[pl.ds((base + c * _CB) * _IPB, _CB * _IPB)],
                idx_v.at[b],
                sem_i[b],
            )

        def wait_out(b):
            pltpu.make_async_copy(
                rows_v.at[b], out_hbm.at[pl.ds(base, _CB)], sem_o[b]
            ).wait()

        def process(c, b, wait_prev_out, prefetch_idx):
            if wait_prev_out:
                wait_out(b)
            pltpu.make_async_copy(
                idx_hbm.at[pl.ds(base * _IPB, _CB * _IPB)],
                idx_v.at[b],
                sem_i[b],
            ).wait()
            copies = []
            for j in range(_CB * _IPB):
                copies.append(
                    pltpu.async_copy(
                        table_hbm.at[idx_v.at[b].at[j]],
                        rows_v.at[b].at[j // _IPB].at[
                            pl.ds((j % _IPB) * _IDXROW, _IDXROW)
                        ],
                        sem_g,
                    )
                )
            for cp in copies:
                cp.wait()
            if prefetch_idx:
                start_idx(c + 2, b)
            pltpu.async_copy(
                rows_v.at[b],
                out_hbm.at[pl.ds(base + c * _CB, _CB)],
                sem_o[b],
            )

        start_idx(0, 0)
        start_idx(1, 1)
        process(0, 0, wait_prev_out=False, prefetch_idx=True)
        process(1, 1, wait_prev_out=False, prefetch_idx=True)

        def body(g, carry):
            c = 2 * g
            process(c, 0, wait_prev_out=True, prefetch_idx=True)
            process(c + 1, 1, wait_prev_out=True, prefetch_idx=True)
            return carry

        lax.fori_loop(1, n_pairs - 1, body, 0)

        c = 2 * (n_pairs - 1)
        process(c, 0, wait_prev_out=True, prefetch_idx=False)
        process(c + 1, 1, wait_prev_out=True, prefetch_idx=False)

        wait_out(0)
        wait_out(1)

    return embed


def kernel(tokens, table):
    idx2d = tokens.reshape(-1, _IDXROW)
    return _make_embed(tokens.shape[0])(table, idx2d)
